# baseline (speedup 1.0000x reference)
import functools

import jax
import jax.numpy as jnp
from jax import lax
from jax.experimental import pallas as pl
from jax.experimental.pallas import tpu as pltpu
from jax.experimental.pallas import tpu_sc as plsc

_B, _S, _H = 4, 2048, 128
_EPS = 1e-5
_NC, _NS = 2, 16
_NW = _NC * _NS
_WPB = _NW // _B
_TPW = _S // _WPB
_CHK = 64
_NCHK = _TPW // _CHK
_NV = _H // 16

_GDN = lax.GatherDimensionNumbers(offset_dims=(), collapsed_slice_dims=(0,),
                                  start_index_map=(0,))


def _lane_sum(x):
    idx = lax.iota(jnp.int32, 16)
    for k in (1, 2, 4, 8):
        perm = jnp.bitwise_xor(idx, jnp.int32(k))
        g = lax.gather(x, perm[:, None], _GDN, (1,),
                       mode=lax.GatherScatterMode.PROMISE_IN_BOUNDS)
        x = x + g
    return x


@functools.cache
def _fused_kernel():
    mesh = plsc.VectorSubcoreMesh(core_axis_name="c", subcore_axis_name="s",
                                  num_cores=_NC, num_subcores=_NS)

    @functools.partial(
        pl.kernel,
        out_type=jax.ShapeDtypeStruct((_B, _S, _H), jnp.float32),
        mesh=mesh,
        scratch_types=[
            pltpu.VMEM((_TPW,), jnp.int32),
            pltpu.VMEM((_TPW, 16), jnp.float32),
            pltpu.VMEM((_TPW, _H), jnp.float32),
            pltpu.VMEM((_TPW, _H), jnp.float32),
            pltpu.VMEM((3, _H), jnp.float32),
            [pltpu.SemaphoreType.DMA] * _NCHK,
            pltpu.SemaphoreType.DMA,
            pltpu.SemaphoreType.DMA,
        ],
    )
    def body(ids_hbm, ttf_hbm, word_hbm, pos2_hbm, dgb_hbm, out_hbm,
             idx_v, ttf_v, rows_v, pos_v, dgb_v, gsems, psem, wsem):
        wid = lax.axis_index("s") * _NC + lax.axis_index("c")
        b = wid // _WPB
        col0 = (wid % _WPB) * _TPW

        pltpu.sync_copy(ids_hbm.at[b, pl.ds(col0, _TPW)], idx_v)
        gcps = [
            pltpu.async_copy(
                word_hbm.at[idx_v.at[pl.ds(q * _CHK, _CHK)]],
                rows_v.at[pl.ds(q * _CHK, _CHK)],
                gsems[q],
            )
            for q in range(_NCHK)
        ]
        pos_cp = pltpu.async_copy(pos2_hbm.at[pl.ds(col0, _TPW)], pos_v, psem)
        pltpu.sync_copy(ttf_hbm.at[pl.ds(wid * _TPW, _TPW)], ttf_v)
        pltpu.sync_copy(dgb_hbm, dgb_v)
        pos_cp.wait()

        half = jnp.float32(0.5)
        three_half = jnp.float32(1.5)
        inv_h = jnp.float32(1.0 / _H)
        seed_a = jnp.float32(14.4)
        seed_b = jnp.float32(0.0173)
        one = jnp.float32(1.0)

        def one_token(i):
            t = ttf_v[i, :]
            e = []
            for j in range(_NV):
                sl = pl.ds(j * 16, 16)
                e.append(rows_v[i, sl] + pos_v[i, sl] + t * dgb_v[0, sl])
            tot = (e[0] + e[1]) + (e[2] + e[3])
            tot = tot + ((e[4] + e[5]) + (e[6] + e[7]))
            sq = (e[0] * e[0] + e[1] * e[1]) + (e[2] * e[2] + e[3] * e[3])
            sq = sq + ((e[4] * e[4] + e[5] * e[5])
                       + (e[6] * e[6] + e[7] * e[7]))
            mean_v = _lane_sum(tot) * inv_h
            msq_v = _lane_sum(sq) * inv_h
            x = msq_v - mean_v * mean_v + _EPS
            y = one / (seed_a * x + seed_b)
            y = y * (three_half - half * x * y * y)
            y = y * (three_half - half * x * y * y)
            y = y * (three_half - half * x * y * y)
            y = y * (three_half - half * x * y * y)
            y = y * (three_half - half * x * y * y)
            for j in range(_NV):
                sl = pl.ds(j * 16, 16)
                rows_v[i, sl] = ((e[j] - mean_v) * y * dgb_v[1, sl]
                                 + dgb_v[2, sl])

        wcps = []
        for q in range(_NCHK):
            gcps[q].wait()

            def pair(i2, carry, _q=q):
                base = _q * _CHK + i2 * 2
                one_token(base)
                one_token(base + 1)
                return carry

            lax.fori_loop(0, _CHK // 2, pair, 0)
            wcps.append(pltpu.async_copy(
                rows_v.at[pl.ds(q * _CHK, _CHK)],
                out_hbm.at[b, pl.ds(col0 + q * _CHK, _CHK)],
                wsem,
            ))
        for c in wcps:
            c.wait()

    return body


def kernel(input_ids, token_type_ids, word_emb, pos_emb, seg_emb, gamma, beta):
    ids = input_ids.astype(jnp.int32)
    ttf = jnp.broadcast_to(
        token_type_ids.astype(jnp.float32).reshape(_B * _S, 1), (_B * _S, 16))
    pos2 = pos_emb + seg_emb[0:1]
    dgb = jnp.stack([seg_emb[1] - seg_emb[0], gamma, beta])
    return _fused_kernel()(ids, ttf, word_emb, pos2, dgb)


# speedup vs baseline: 1.0644x; 1.0644x over previous
import functools

import jax
import jax.numpy as jnp
from jax import lax
from jax.experimental import pallas as pl
from jax.experimental.pallas import tpu as pltpu
from jax.experimental.pallas import tpu_sc as plsc

_B, _S, _H = 4, 2048, 128
_EPS = 1e-5
_NC, _NS = 2, 16
_NW = _NC * _NS
_WPB = _NW // _B
_TPW = _S // _WPB
_CHK = 64
_NCHK = _TPW // _CHK
_NV = _H // 16

_GDN = lax.GatherDimensionNumbers(offset_dims=(), collapsed_slice_dims=(0,),
                                  start_index_map=(0,))


def _lane_sum(x):
    idx = lax.iota(jnp.int32, 16)
    for k in (1, 2, 4, 8):
        perm = jnp.bitwise_xor(idx, jnp.int32(k))
        g = lax.gather(x, perm[:, None], _GDN, (1,),
                       mode=lax.GatherScatterMode.PROMISE_IN_BOUNDS)
        x = x + g
    return x


@functools.cache
def _fused_kernel():
    mesh = plsc.VectorSubcoreMesh(core_axis_name="c", subcore_axis_name="s",
                                  num_cores=_NC, num_subcores=_NS)

    @functools.partial(
        pl.kernel,
        out_type=jax.ShapeDtypeStruct((_B, _S, _H), jnp.float32),
        mesh=mesh,
        scratch_types=[
            pltpu.VMEM((_TPW,), jnp.int32),
            pltpu.VMEM((_TPW, 16), jnp.float32),
            pltpu.VMEM((_TPW, _H), jnp.float32),
            pltpu.VMEM((_TPW, _H), jnp.float32),
            pltpu.VMEM((3, _H), jnp.float32),
            [pltpu.SemaphoreType.DMA] * _NCHK,
            pltpu.SemaphoreType.DMA,
            pltpu.SemaphoreType.DMA,
        ],
    )
    def body(ids_hbm, ttf_hbm, word_hbm, pos2_hbm, dgb_hbm, out_hbm,
             idx_v, ttf_v, rows_v, pos_v, dgb_v, gsems, psem, wsem):
        wid = lax.axis_index("s") * _NC + lax.axis_index("c")
        b = wid // _WPB
        col0 = (wid % _WPB) * _TPW

        pltpu.sync_copy(ids_hbm.at[b, pl.ds(col0, _TPW)], idx_v)
        gcps = [
            pltpu.async_copy(
                word_hbm.at[idx_v.at[pl.ds(q * _CHK, _CHK)]],
                rows_v.at[pl.ds(q * _CHK, _CHK)],
                gsems[q],
            )
            for q in range(_NCHK)
        ]
        pos_cp = pltpu.async_copy(pos2_hbm.at[pl.ds(col0, _TPW)], pos_v, psem)
        pltpu.sync_copy(ttf_hbm.at[pl.ds(wid * _TPW, _TPW)], ttf_v)
        pltpu.sync_copy(dgb_hbm, dgb_v)
        pos_cp.wait()

        half = jnp.float32(0.5)
        three_half = jnp.float32(1.5)
        inv_h = jnp.float32(1.0 / _H)
        seed_a = jnp.float32(14.4)
        seed_b = jnp.float32(0.0173)
        one = jnp.float32(1.0)

        def one_token(i):
            t = ttf_v[i, :]
            e = []
            for j in range(_NV):
                sl = pl.ds(j * 16, 16)
                e.append(rows_v[i, sl] + pos_v[i, sl] + t * dgb_v[0, sl])
            tot = (e[0] + e[1]) + (e[2] + e[3])
            tot = tot + ((e[4] + e[5]) + (e[6] + e[7]))
            sq = (e[0] * e[0] + e[1] * e[1]) + (e[2] * e[2] + e[3] * e[3])
            sq = sq + ((e[4] * e[4] + e[5] * e[5])
                       + (e[6] * e[6] + e[7] * e[7]))
            mean_v = _lane_sum(tot) * inv_h
            msq_v = _lane_sum(sq) * inv_h
            x = msq_v - mean_v * mean_v + _EPS
            y = one / (seed_a * x + seed_b)
            y = y * (three_half - half * x * y * y)
            y = y * (three_half - half * x * y * y)
            y = y * (three_half - half * x * y * y)
            y = y * (three_half - half * x * y * y)
            y = y * (three_half - half * x * y * y)
            for j in range(_NV):
                sl = pl.ds(j * 16, 16)
                rows_v[i, sl] = ((e[j] - mean_v) * y * dgb_v[1, sl]
                                 + dgb_v[2, sl])

        wcps = []
        for q in range(_NCHK):
            gcps[q].wait()

            @plsc.parallel_loop(q * _CHK, (q + 1) * _CHK, step=1, unroll=4)
            def _(i):
                one_token(i)
            wcps.append(pltpu.async_copy(
                rows_v.at[pl.ds(q * _CHK, _CHK)],
                out_hbm.at[b, pl.ds(col0 + q * _CHK, _CHK)],
                wsem,
            ))
        for c in wcps:
            c.wait()

    return body


def kernel(input_ids, token_type_ids, word_emb, pos_emb, seg_emb, gamma, beta):
    ids = input_ids.astype(jnp.int32)
    ttf = jnp.broadcast_to(
        token_type_ids.astype(jnp.float32).reshape(_B * _S, 1), (_B * _S, 16))
    pos2 = pos_emb + seg_emb[0:1]
    dgb = jnp.stack([seg_emb[1] - seg_emb[0], gamma, beta])
    return _fused_kernel()(ids, ttf, word_emb, pos2, dgb)


# speedup vs baseline: 1.8951x; 1.7804x over previous
import functools

import jax
import jax.numpy as jnp
from jax import lax
from jax.experimental import pallas as pl
from jax.experimental.pallas import tpu as pltpu
from jax.experimental.pallas import tpu_sc as plsc

_B, _S, _H = 4, 2048, 128
_N = _B * _S
_EPS = 1e-5
_NC, _NS = 2, 16
_NW = _NC * _NS
_WPB = _NW // _B
_TPW = _S // _WPB
_CHK = 64
_NCHK = _TPW // _CHK


@functools.cache
def _gather_words_kernel():
    mesh = plsc.VectorSubcoreMesh(core_axis_name="c", subcore_axis_name="s",
                                  num_cores=_NC, num_subcores=_NS)

    @functools.partial(
        pl.kernel,
        out_type=jax.ShapeDtypeStruct((_N, _H), jnp.float32),
        mesh=mesh,
        scratch_types=[
            pltpu.VMEM((_TPW,), jnp.int32),
            pltpu.VMEM((_TPW, _H), jnp.float32),
            [pltpu.SemaphoreType.DMA] * _NCHK,
            pltpu.SemaphoreType.DMA,
        ],
    )
    def body(ids_hbm, word_hbm, out_hbm, idx_v, rows_v, gsems, wsem):
        wid = lax.axis_index("s") * _NC + lax.axis_index("c")
        b = wid // _WPB
        col0 = (wid % _WPB) * _TPW

        pltpu.sync_copy(ids_hbm.at[b, pl.ds(col0, _TPW)], idx_v)
        gcps = [
            pltpu.async_copy(
                word_hbm.at[idx_v.at[pl.ds(q * _CHK, _CHK)]],
                rows_v.at[pl.ds(q * _CHK, _CHK)],
                gsems[q],
            )
            for q in range(_NCHK)
        ]
        wcps = []
        for q in range(_NCHK):
            gcps[q].wait()
            wcps.append(pltpu.async_copy(
                rows_v.at[pl.ds(q * _CHK, _CHK)],
                out_hbm.at[pl.ds(wid * _TPW + q * _CHK, _CHK)],
                wsem,
            ))
        for c in wcps:
            c.wait()

    return body


_BLK = 4096


def _add_ln_body(x_ref, pos_ref, ttf_ref, seg_ref, gam_ref, bet_ref, o_ref):
    s0 = seg_ref[0:1, :]
    dseg = seg_ref[1:2, :] - s0
    x = x_ref[...].reshape(_BLK // _S, _S, _H) + pos_ref[...][None]
    x = x.reshape(_BLK, _H) + s0 + ttf_ref[...] * dseg
    mean = jnp.mean(x, axis=-1, keepdims=True)
    xc = x - mean
    var = jnp.mean(xc * xc, axis=-1, keepdims=True)
    o_ref[...] = xc * lax.rsqrt(var + _EPS) * gam_ref[...] + bet_ref[...]


def _add_ln(gathered, pos_emb, ttf, seg_emb, gamma, beta):
    return pl.pallas_call(
        _add_ln_body,
        grid=(_N // _BLK,),
        in_specs=[
            pl.BlockSpec((_BLK, _H), lambda i: (i, 0)),
            pl.BlockSpec((_S, _H), lambda i: (0, 0)),
            pl.BlockSpec((_BLK, 1), lambda i: (i, 0)),
            pl.BlockSpec((2, _H), lambda i: (0, 0)),
            pl.BlockSpec((1, _H), lambda i: (0, 0)),
            pl.BlockSpec((1, _H), lambda i: (0, 0)),
        ],
        out_specs=pl.BlockSpec((_BLK, _H), lambda i: (i, 0)),
        out_shape=jax.ShapeDtypeStruct((_N, _H), jnp.float32),
    )(gathered, pos_emb, ttf, seg_emb, gamma, beta)


def kernel(input_ids, token_type_ids, word_emb, pos_emb, seg_emb, gamma, beta):
    ids = input_ids.astype(jnp.int32)
    gathered = _gather_words_kernel()(ids, word_emb)
    ttf = token_type_ids.astype(jnp.float32).reshape(_N, 1)
    out = _add_ln(gathered, pos_emb, ttf, seg_emb,
                  gamma.reshape(1, _H), beta.reshape(1, _H))
    return out.reshape(_B, _S, _H)
